# baseline (speedup 1.0000x reference)
import jax
import jax.numpy as jnp
from jax.experimental import pallas as pl

ALPHA = 0.2
NHEADS = 4


def _norm_softmax_body(num_ref, den_ref, o_ref):
    h = num_ref[...] / (den_ref[...] + 1e-16)
    m = jnp.max(h, axis=1, keepdims=True)
    ex = jnp.exp(h - m)
    o_ref[...] = ex / jnp.sum(ex, axis=1, keepdims=True)


def _norm_softmax(num, den):
    N, F = num.shape
    B = 2000
    return pl.pallas_call(
        _norm_softmax_body,
        grid=(N // B,),
        in_specs=[
            pl.BlockSpec((B, F), lambda i: (i, 0)),
            pl.BlockSpec((B, 1), lambda i: (i, 0)),
        ],
        out_specs=pl.BlockSpec((B, F), lambda i: (i, 0)),
        out_shape=jax.ShapeDtypeStruct((N, F), jnp.float32),
    )(num, den)


def kernel(x, edges, W_lin, b_lin, W_heads, a_heads, W_end, a_end):
    src = edges[0]
    dst = edges[1]
    N = x.shape[0]
    F = x.shape[1]
    h = x @ W_lin + b_lin

    heads = []
    for i in range(NHEADS):
        hi = h @ W_heads[i]
        sl = hi @ a_heads[i][0, :F]
        sr = hi @ a_heads[i][0, F:]
        e = jnp.exp(-jax.nn.leaky_relu(sl[src] + sr[dst], ALPHA))
        rs = jax.ops.segment_sum(e, src, num_segments=N)
        hp = jax.ops.segment_sum(e[:, None] * hi[dst], src, num_segments=N)
        heads.append(jax.nn.elu(hp / (rs[:, None] + 1e-16)))

    hcat = jnp.concatenate(heads, axis=1)
    h2 = hcat @ W_end
    F2 = h2.shape[1]
    sl = h2 @ a_end[0, :F2]
    sr = h2 @ a_end[0, F2:]
    e = jnp.exp(-jax.nn.leaky_relu(sl[src] + sr[dst], ALPHA))
    rs = jax.ops.segment_sum(e, src, num_segments=N)
    num = jax.ops.segment_sum(e[:, None] * h2[dst], src, num_segments=N)
    return _norm_softmax(num, rs[:, None])


# speedup vs baseline: 12.3501x; 12.3501x over previous
import functools

import jax
import jax.numpy as jnp
from jax import lax
from jax.experimental import pallas as pl
from jax.experimental.pallas import tpu as pltpu
from jax.experimental.pallas import tpu_sc as plsc

ALPHA = 0.2
N_HEADS = 4
F = 128
FE = 144
N_PAD = 10240
C = 80
NW = 32


def _proj1_body(x_ref, wl_ref, bl_ref, wh_ref, h_ref, hx0, hx1, hx2, hx3):
    h = x_ref[...] @ wl_ref[...] + bl_ref[...][None, :]
    h_ref[...] = h
    bn = h.shape[0]
    ones_col = (lax.broadcasted_iota(jnp.int32, (bn, FE - F), 1) == 0).astype(jnp.float32)
    outs = (hx0, hx1, hx2, hx3)
    for i in range(N_HEADS):
        hi = h @ wh_ref[i]
        outs[i][...] = jnp.concatenate([hi, ones_col], axis=1)


def _proj1(x, w_lin, b_lin, w_heads):
    n = x.shape[0]
    bn = 2000
    grid = (n // bn,)
    return pl.pallas_call(
        _proj1_body,
        grid=grid,
        in_specs=[
            pl.BlockSpec((bn, F), lambda i: (i, 0)),
            pl.BlockSpec((F, F), lambda i: (0, 0)),
            pl.BlockSpec((F,), lambda i: (0,)),
            pl.BlockSpec((N_HEADS, F, F), lambda i: (0, 0, 0)),
        ],
        out_specs=[pl.BlockSpec((bn, F), lambda i: (i, 0))]
        + [pl.BlockSpec((bn, FE), lambda i: (i, 0)) for _ in range(N_HEADS)],
        out_shape=[jax.ShapeDtypeStruct((n, F), jnp.float32)]
        + [jax.ShapeDtypeStruct((n, FE), jnp.float32) for _ in range(N_HEADS)],
    )(x, w_lin, b_lin, w_heads)


def _avec1_body(h_ref, wh_ref, ah_ref, out_ref):
    h = h_ref[...]
    for i in range(N_HEADS):
        a = ah_ref[i, 0, :]
        v_l = jnp.sum(wh_ref[i] * a[:F][None, :], axis=1)
        v_r = jnp.sum(wh_ref[i] * a[F:][None, :], axis=1)
        out_ref[i, :] = jnp.sum(h * v_l[None, :], axis=1)
        out_ref[N_HEADS + i, :] = jnp.sum(h * v_r[None, :], axis=1)


def _avec1(h, w_heads, a_heads):
    n = h.shape[0]
    return pl.pallas_call(
        _avec1_body,
        out_shape=jax.ShapeDtypeStruct((2 * N_HEADS, n), jnp.float32),
    )(h, w_heads, a_heads)


def _mid_body(p0, p1, p2, p3, we_ref, out_ref):
    cols = []
    for p in (p0, p1, p2, p3):
        s = p[0] + p[1]
        num = s[:, :F]
        den = jnp.sum(s[:, F:], axis=1)
        hp = num / (den[:, None] + 1e-16)
        cols.append(jnp.where(hp > 0, hp, jnp.exp(jnp.minimum(hp, 0.0)) - 1.0))
    hcat = jnp.concatenate(cols, axis=1)
    h2 = hcat @ we_ref[...]
    bn = h2.shape[0]
    ones_col = (lax.broadcasted_iota(jnp.int32, (bn, FE - F), 1) == 0).astype(jnp.float32)
    out_ref[...] = jnp.concatenate([h2, ones_col], axis=1)


def _mid(parts, w_end, n):
    bn = 2000
    grid = (n // bn,)
    pspec = pl.BlockSpec((2, bn, FE), lambda i: (0, i, 0))
    return pl.pallas_call(
        _mid_body,
        grid=grid,
        in_specs=[pspec, pspec, pspec, pspec,
                  pl.BlockSpec((4 * F, F), lambda i: (0, 0))],
        out_specs=pl.BlockSpec((bn, FE), lambda i: (i, 0)),
        out_shape=jax.ShapeDtypeStruct((n, FE), jnp.float32),
    )(*parts, w_end)


def _avec2_body(h_ref, a_ref, out_ref):
    h = h_ref[...][:, :F]
    a = a_ref[0]
    out_ref[0, :] = jnp.sum(h * a[:F][None, :], axis=1)
    out_ref[1, :] = jnp.sum(h * a[F:][None, :], axis=1)


def _avec2(h2ext, a_end):
    n = h2ext.shape[0]
    return pl.pallas_call(
        _avec2_body,
        out_shape=jax.ShapeDtypeStruct((2, n), jnp.float32),
    )(h2ext, a_end)


def _final_body(p_ref, out_ref):
    s = p_ref[0] + p_ref[1]
    num = s[:, :F]
    den = jnp.sum(s[:, F:], axis=1)
    hp = num / (den[:, None] + 1e-16)
    m = jnp.max(hp, axis=1, keepdims=True)
    ex = jnp.exp(hp - m)
    out_ref[...] = ex / jnp.sum(ex, axis=1, keepdims=True)


def _final(part, n):
    bn = 2000
    grid = (n // bn,)
    return pl.pallas_call(
        _final_body,
        grid=grid,
        in_specs=[pl.BlockSpec((2, bn, FE), lambda i: (0, i, 0))],
        out_specs=pl.BlockSpec((bn, F), lambda i: (i, 0)),
        out_shape=jax.ShapeDtypeStruct((n, F), jnp.float32),
    )(part)


_MESH = plsc.VectorSubcoreMesh(
    core_axis_name="c", subcore_axis_name="s", num_cores=2, num_subcores=16)


def _sc_pass(h_ext, sl, sr, src, dst, zeros):
    n = h_ext.shape[0]
    e = src.shape[0]
    epw = e // NW
    nchunk = epw // C
    rpt = N_PAD // 16

    def body(h_hbm, sl_hbm, sr_hbm, src_hbm, dst_hbm, z_hbm, out_hbm,
             sl_v, sr_v, srcv, dstv, ev, rows, acc, sem):
        cid = lax.axis_index("c")
        sid = lax.axis_index("s")
        wid = cid * 16 + sid
        pltpu.sync_copy(sl_hbm, sl_v)
        pltpu.sync_copy(sr_hbm, sr_v)
        pltpu.sync_copy(z_hbm, acc.at[pl.ds(sid * rpt, rpt)])
        plsc.subcore_barrier()

        def chunk(c, carry):
            base = wid * epw + c * C
            pltpu.sync_copy(src_hbm.at[pl.ds(base, C)], srcv)
            pltpu.sync_copy(dst_hbm.at[pl.ds(base, C)], dstv)
            gat = pltpu.async_copy(h_hbm.at[dstv], rows, sem)
            for g in range(C // 16):
                si = srcv[pl.ds(g * 16, 16)]
                di = dstv[pl.ds(g * 16, 16)]
                lg = plsc.load_gather(sl_v, [si]) + plsc.load_gather(sr_v, [di])
                lr = jnp.where(lg > 0, lg, lg * ALPHA)
                ev[pl.ds(g * 16, 16)] = jnp.exp(-lr)
            gat.wait()

            def scale(k, carry2):
                es = plsc.load_gather(ev, [jnp.full((16,), k, jnp.int32)])
                for j in range(FE // 16):
                    rows[k, pl.ds(j * 16, 16)] = rows[k, pl.ds(j * 16, 16)] * es
                return carry2

            lax.fori_loop(0, C, scale, 0)
            pltpu.sync_copy(rows, acc.at[srcv], add=True)
            return carry

        lax.fori_loop(0, nchunk, chunk, 0)
        plsc.subcore_barrier()
        for k in range(rpt // C):
            pltpu.sync_copy(acc.at[pl.ds(sid * rpt + k * C, C)],
                            out_hbm.at[cid, pl.ds(sid * rpt + k * C, C)])

    run = pl.kernel(
        body,
        out_type=jax.ShapeDtypeStruct((2, N_PAD, FE), jnp.float32),
        mesh=_MESH,
        scratch_types=[
            pltpu.VMEM((n,), jnp.float32),
            pltpu.VMEM((n,), jnp.float32),
            pltpu.VMEM((C,), jnp.int32),
            pltpu.VMEM((C,), jnp.int32),
            pltpu.VMEM((C,), jnp.float32),
            pltpu.VMEM((C, FE), jnp.float32),
            pltpu.VMEM_SHARED((N_PAD, FE), jnp.float32),
            pltpu.SemaphoreType.DMA,
        ],
        compiler_params=pltpu.CompilerParams(
            needs_layout_passes=False, use_tc_tiling_on_sc=False),
    )
    return run(h_ext, sl, sr, src, dst, zeros)


def kernel(x, edges, W_lin, b_lin, W_heads, a_heads, W_end, a_end):
    n = x.shape[0]
    src = edges[0]
    dst = edges[1]
    zeros = jnp.zeros((N_PAD // 16, FE), jnp.float32)

    h, hx0, hx1, hx2, hx3 = _proj1(x, W_lin, b_lin, W_heads)
    slsr = _avec1(h, W_heads, a_heads)
    parts = [
        _sc_pass(hx, slsr[i], slsr[N_HEADS + i], src, dst, zeros)
        for i, hx in enumerate((hx0, hx1, hx2, hx3))
    ]
    h2ext = _mid(parts, W_end, n)
    slsr2 = _avec2(h2ext, a_end)
    part2 = _sc_pass(h2ext, slsr2[0], slsr2[1], src, dst, zeros)
    return _final(part2, n)


# speedup vs baseline: 24.4600x; 1.9805x over previous
import jax
import jax.numpy as jnp
from jax import lax
from jax.experimental import pallas as pl
from jax.experimental.pallas import tpu as pltpu
from jax.experimental.pallas import tpu_sc as plsc

ALPHA = 0.2
N_HEADS = 4
F = 128
N_PAD = 10240
C = 80
NW = 32
IB = 25


def _proj1_body(x_ref, wl_ref, bl_ref, wh_ref, h_ref, hx0, hx1, hx2, hx3):
    h = x_ref[...] @ wl_ref[...] + bl_ref[...][None, :]
    h_ref[...] = h
    outs = (hx0, hx1, hx2, hx3)
    for i in range(N_HEADS):
        outs[i][...] = h @ wh_ref[i]


def _proj1(x, w_lin, b_lin, w_heads):
    n = x.shape[0]
    bn = 2000
    grid = (n // bn,)
    return pl.pallas_call(
        _proj1_body,
        grid=grid,
        in_specs=[
            pl.BlockSpec((bn, F), lambda i: (i, 0)),
            pl.BlockSpec((F, F), lambda i: (0, 0)),
            pl.BlockSpec((F,), lambda i: (0,)),
            pl.BlockSpec((N_HEADS, F, F), lambda i: (0, 0, 0)),
        ],
        out_specs=[pl.BlockSpec((bn, F), lambda i: (i, 0)) for _ in range(5)],
        out_shape=[jax.ShapeDtypeStruct((n, F), jnp.float32) for _ in range(5)],
    )(x, w_lin, b_lin, w_heads)


def _avec1_body(h_ref, wh_ref, ah_ref, out_ref):
    h = h_ref[...]
    for i in range(N_HEADS):
        a = ah_ref[i, 0, :]
        v_l = jnp.sum(wh_ref[i] * a[:F][None, :], axis=1)
        v_r = jnp.sum(wh_ref[i] * a[F:][None, :], axis=1)
        out_ref[i, :] = jnp.sum(h * v_l[None, :], axis=1)
        out_ref[N_HEADS + i, :] = jnp.sum(h * v_r[None, :], axis=1)


def _avec1(h, w_heads, a_heads):
    n = h.shape[0]
    return pl.pallas_call(
        _avec1_body,
        out_shape=jax.ShapeDtypeStruct((2 * N_HEADS, n), jnp.float32),
    )(h, w_heads, a_heads)


def _mid_body(p0, p1, p2, p3, r0, r1, r2, r3, we_ref, out_ref):
    cols = []
    for p, r in zip((p0, p1, p2, p3), (r0, r1, r2, r3)):
        num = p[0] + p[1]
        den = r[0, :, 0] + r[1, :, 0]
        hp = num / (den[:, None] + 1e-16)
        cols.append(jnp.where(hp > 0, hp, jnp.exp(jnp.minimum(hp, 0.0)) - 1.0))
    hcat = jnp.concatenate(cols, axis=1)
    out_ref[...] = hcat @ we_ref[...]


def _mid(parts, rsums, w_end, n):
    bn = 2000
    grid = (n // bn,)
    pspec = pl.BlockSpec((2, bn, F), lambda i: (0, i, 0))
    rspec = pl.BlockSpec((2, bn, 1), lambda i: (0, i, 0))
    return pl.pallas_call(
        _mid_body,
        grid=grid,
        in_specs=[pspec] * 4 + [rspec] * 4
        + [pl.BlockSpec((4 * F, F), lambda i: (0, 0))],
        out_specs=pl.BlockSpec((bn, F), lambda i: (i, 0)),
        out_shape=jax.ShapeDtypeStruct((n, F), jnp.float32),
    )(*parts, *rsums, w_end)


def _avec2_body(h_ref, a_ref, out_ref):
    h = h_ref[...]
    a = a_ref[0]
    out_ref[0, :] = jnp.sum(h * a[:F][None, :], axis=1)
    out_ref[1, :] = jnp.sum(h * a[F:][None, :], axis=1)


def _avec2(h2, a_end):
    n = h2.shape[0]
    return pl.pallas_call(
        _avec2_body,
        out_shape=jax.ShapeDtypeStruct((2, n), jnp.float32),
    )(h2, a_end)


def _final_body(p_ref, r_ref, out_ref):
    num = p_ref[0] + p_ref[1]
    den = r_ref[0, :, 0] + r_ref[1, :, 0]
    hp = num / (den[:, None] + 1e-16)
    m = jnp.max(hp, axis=1, keepdims=True)
    ex = jnp.exp(hp - m)
    out_ref[...] = ex / jnp.sum(ex, axis=1, keepdims=True)


def _final(part, rsum, n):
    bn = 2000
    grid = (n // bn,)
    return pl.pallas_call(
        _final_body,
        grid=grid,
        in_specs=[pl.BlockSpec((2, bn, F), lambda i: (0, i, 0)),
                  pl.BlockSpec((2, bn, 1), lambda i: (0, i, 0))],
        out_specs=pl.BlockSpec((bn, F), lambda i: (i, 0)),
        out_shape=jax.ShapeDtypeStruct((n, F), jnp.float32),
    )(part, rsum)


_MESH = plsc.VectorSubcoreMesh(
    core_axis_name="c", subcore_axis_name="s", num_cores=2, num_subcores=16)


def _sc_pass(h_mat, sl, sr, src2, dst2, zrow, zrs):
    n = h_mat.shape[0]
    e = src2.shape[0] * src2.shape[1]
    cpw = e // (NW * C)
    nib = cpw // IB
    rpt = N_PAD // 16

    def body(h_hbm, sl_hbm, sr_hbm, src_hbm, dst_hbm, z_hbm, zr_hbm,
             out_hbm, outrs_hbm,
             sl_v, sr_v, srcb, dstb, ev, rows0, rows1, acc, rs, g0, g1):
        cid = lax.axis_index("c")
        sid = lax.axis_index("s")
        wid = cid * 16 + sid
        pltpu.sync_copy(sl_hbm, sl_v)
        pltpu.sync_copy(sr_hbm, sr_v)
        pltpu.sync_copy(z_hbm, acc.at[pl.ds(sid * rpt, rpt)])
        pltpu.sync_copy(zr_hbm, rs.at[pl.ds(sid * rpt, rpt)])
        plsc.subcore_barrier()

        def step(c, buf, gsem, nbuf, ngsem, prefetch):
            if prefetch:
                pltpu.async_copy(h_hbm.at[dstb.at[c + 1]], nbuf, ngsem)
            for g in range(C // 16):
                si = srcb[c, pl.ds(g * 16, 16)]
                di = dstb[c, pl.ds(g * 16, 16)]
                lg = plsc.load_gather(sl_v, [si]) + plsc.load_gather(sr_v, [di])
                lr = jnp.where(lg > 0, lg, lg * ALPHA)
                ev[pl.ds(g * 16, 16)] = jnp.exp(-lr)
            pltpu.make_async_copy(h_hbm.at[dstb.at[c]], buf, gsem).wait()

            def scale(k, carry2):
                es = plsc.load_gather(ev, [jnp.full((16,), k, jnp.int32)])
                for j in range(F // 16):
                    buf[k, pl.ds(j * 16, 16)] = buf[k, pl.ds(j * 16, 16)] * es
                return carry2

            lax.fori_loop(0, C, scale, 0, unroll=8)
            pltpu.sync_copy(buf, acc.at[srcb.at[c]], add=True)
            pltpu.sync_copy(ev, rs.at[srcb.at[c]], add=True)

        def ib_body(ib, carry):
            rbase = wid * cpw + ib * IB
            pltpu.sync_copy(src_hbm.at[pl.ds(rbase, IB)], srcb)
            pltpu.sync_copy(dst_hbm.at[pl.ds(rbase, IB)], dstb)
            pltpu.async_copy(h_hbm.at[dstb.at[0]], rows0, g0)

            def pair(j, carry2):
                c0 = 2 * j
                step(c0, rows0, g0, rows1, g1, True)
                step(c0 + 1, rows1, g1, rows0, g0, True)
                return carry2

            lax.fori_loop(0, (IB - 1) // 2, pair, 0)
            step(IB - 1, rows0, g0, rows1, g1, False)
            return carry

        lax.fori_loop(0, nib, ib_body, 0)
        plsc.subcore_barrier()
        for k in range(rpt // C):
            pltpu.sync_copy(acc.at[pl.ds(sid * rpt + k * C, C)],
                            out_hbm.at[cid, pl.ds(sid * rpt + k * C, C)])
        pltpu.sync_copy(rs.at[pl.ds(sid * rpt, rpt)],
                        outrs_hbm.at[cid, pl.ds(sid * rpt, rpt)])

    run = pl.kernel(
        body,
        out_type=(jax.ShapeDtypeStruct((2, N_PAD, F), jnp.float32),
                  jax.ShapeDtypeStruct((2, N_PAD), jnp.float32)),
        mesh=_MESH,
        scratch_types=[
            pltpu.VMEM((n,), jnp.float32),
            pltpu.VMEM((n,), jnp.float32),
            pltpu.VMEM((IB, C), jnp.int32),
            pltpu.VMEM((IB, C), jnp.int32),
            pltpu.VMEM((C,), jnp.float32),
            pltpu.VMEM((C, F), jnp.float32),
            pltpu.VMEM((C, F), jnp.float32),
            pltpu.VMEM_SHARED((N_PAD, F), jnp.float32),
            pltpu.VMEM_SHARED((N_PAD,), jnp.float32),
            pltpu.SemaphoreType.DMA,
            pltpu.SemaphoreType.DMA,
        ],
        compiler_params=pltpu.CompilerParams(
            needs_layout_passes=False, use_tc_tiling_on_sc=False),
    )
    return run(h_mat, sl, sr, src2, dst2, zrow, zrs)


def kernel(x, edges, W_lin, b_lin, W_heads, a_heads, W_end, a_end):
    n = x.shape[0]
    src2 = edges[0].reshape(-1, C)
    dst2 = edges[1].reshape(-1, C)
    zrow = jnp.zeros((N_PAD // 16, F), jnp.float32)
    zrs = jnp.zeros((N_PAD // 16,), jnp.float32)

    h, h0, h1, h2_, h3 = _proj1(x, W_lin, b_lin, W_heads)
    slsr = _avec1(h, W_heads, a_heads)
    parts = []
    rsums = []
    for i, hx in enumerate((h0, h1, h2_, h3)):
        p, r = _sc_pass(hx, slsr[i], slsr[N_HEADS + i], src2, dst2, zrow, zrs)
        parts.append(p)
        rsums.append(r.reshape(2, N_PAD, 1))
    h2 = _mid(parts, rsums, W_end, n)
    slsr2 = _avec2(h2, a_end)
    p2, r2 = _sc_pass(h2, slsr2[0], slsr2[1], src2, dst2, zrow, zrs)
    return _final(p2, r2.reshape(2, N_PAD, 1), n)


# speedup vs baseline: 24.9305x; 1.0192x over previous
import jax
import jax.numpy as jnp
from jax import lax
from jax.experimental import pallas as pl
from jax.experimental.pallas import tpu as pltpu
from jax.experimental.pallas import tpu_sc as plsc

ALPHA = 0.2
N_HEADS = 4
F = 128
N_PAD = 10240
C = 80
NW = 32
IB = 25


def _proj1_body(x_ref, wl_ref, bl_ref, wh_ref, h_ref, hx0, hx1, hx2, hx3):
    h = x_ref[...] @ wl_ref[...] + bl_ref[...][None, :]
    h_ref[...] = h
    outs = (hx0, hx1, hx2, hx3)
    for i in range(N_HEADS):
        outs[i][...] = h @ wh_ref[i]


def _proj1(x, w_lin, b_lin, w_heads):
    n = x.shape[0]
    bn = 2000
    grid = (n // bn,)
    return pl.pallas_call(
        _proj1_body,
        grid=grid,
        in_specs=[
            pl.BlockSpec((bn, F), lambda i: (i, 0)),
            pl.BlockSpec((F, F), lambda i: (0, 0)),
            pl.BlockSpec((F,), lambda i: (0,)),
            pl.BlockSpec((N_HEADS, F, F), lambda i: (0, 0, 0)),
        ],
        out_specs=[pl.BlockSpec((bn, F), lambda i: (i, 0)) for _ in range(5)],
        out_shape=[jax.ShapeDtypeStruct((n, F), jnp.float32) for _ in range(5)],
    )(x, w_lin, b_lin, w_heads)


def _avec1_body(h_ref, wh_ref, ah_ref, out_ref):
    h = h_ref[...]
    for i in range(N_HEADS):
        a = ah_ref[i, 0, :]
        v_l = jnp.sum(wh_ref[i] * a[:F][None, :], axis=1)
        v_r = jnp.sum(wh_ref[i] * a[F:][None, :], axis=1)
        out_ref[i, :] = jnp.sum(h * v_l[None, :], axis=1)
        out_ref[N_HEADS + i, :] = jnp.sum(h * v_r[None, :], axis=1)


def _avec1(h, w_heads, a_heads):
    n = h.shape[0]
    return pl.pallas_call(
        _avec1_body,
        out_shape=jax.ShapeDtypeStruct((2 * N_HEADS, n), jnp.float32),
    )(h, w_heads, a_heads)


def _mid_body(p0, p1, p2, p3, r0, r1, r2, r3, we_ref, out_ref):
    cols = []
    for p, r in zip((p0, p1, p2, p3), (r0, r1, r2, r3)):
        num = p[0] + p[1]
        den = r[0, :, 0] + r[1, :, 0]
        hp = num / (den[:, None] + 1e-16)
        cols.append(jnp.where(hp > 0, hp, jnp.exp(jnp.minimum(hp, 0.0)) - 1.0))
    hcat = jnp.concatenate(cols, axis=1)
    out_ref[...] = hcat @ we_ref[...]


def _mid(parts, rsums, w_end, n):
    bn = 2000
    grid = (n // bn,)
    pspec = pl.BlockSpec((2, bn, F), lambda i: (0, i, 0))
    rspec = pl.BlockSpec((2, bn, 1), lambda i: (0, i, 0))
    return pl.pallas_call(
        _mid_body,
        grid=grid,
        in_specs=[pspec] * 4 + [rspec] * 4
        + [pl.BlockSpec((4 * F, F), lambda i: (0, 0))],
        out_specs=pl.BlockSpec((bn, F), lambda i: (i, 0)),
        out_shape=jax.ShapeDtypeStruct((n, F), jnp.float32),
    )(*parts, *rsums, w_end)


def _avec2_body(h_ref, a_ref, out_ref):
    h = h_ref[...]
    a = a_ref[0]
    out_ref[0, :] = jnp.sum(h * a[:F][None, :], axis=1)
    out_ref[1, :] = jnp.sum(h * a[F:][None, :], axis=1)


def _avec2(h2, a_end):
    n = h2.shape[0]
    return pl.pallas_call(
        _avec2_body,
        out_shape=jax.ShapeDtypeStruct((2, n), jnp.float32),
    )(h2, a_end)


def _final_body(p_ref, r_ref, out_ref):
    num = p_ref[0] + p_ref[1]
    den = r_ref[0, :, 0] + r_ref[1, :, 0]
    hp = num / (den[:, None] + 1e-16)
    m = jnp.max(hp, axis=1, keepdims=True)
    ex = jnp.exp(hp - m)
    out_ref[...] = ex / jnp.sum(ex, axis=1, keepdims=True)


def _final(part, rsum, n):
    bn = 2000
    grid = (n // bn,)
    return pl.pallas_call(
        _final_body,
        grid=grid,
        in_specs=[pl.BlockSpec((2, bn, F), lambda i: (0, i, 0)),
                  pl.BlockSpec((2, bn, 1), lambda i: (0, i, 0))],
        out_specs=pl.BlockSpec((bn, F), lambda i: (i, 0)),
        out_shape=jax.ShapeDtypeStruct((n, F), jnp.float32),
    )(part, rsum)


_MESH = plsc.VectorSubcoreMesh(
    core_axis_name="c", subcore_axis_name="s", num_cores=2, num_subcores=16)


def _sc_pass(h_mat, sl, sr, src2, dst2, zrow, zrs, zidx):
    n = h_mat.shape[0]
    e = src2.shape[0] * src2.shape[1]
    cpw = e // (NW * C)
    nib = cpw // IB
    rpt = N_PAD // 16
    ntrip = (IB - 4) // 3

    def body(h_hbm, sl_hbm, sr_hbm, src_hbm, dst_hbm, z_hbm, zr_hbm, zi_hbm,
             out_hbm, outrs_hbm,
             srcb, dstb, r0, r1, r2, sl0, sl1, sl2, sr0, sr1, sr2,
             e0, e1, e2, ziv, acc, rs,
             g0, g1, g2, s0, s1, s2, q0, q1, q2):
        cid = lax.axis_index("c")
        sid = lax.axis_index("s")
        wid = cid * 16 + sid
        rows = (r0, r1, r2)
        slv = (sl0, sl1, sl2)
        srv = (sr0, sr1, sr2)
        ev = (e0, e1, e2)
        gsem = (g0, g1, g2)
        ssem = (s0, s1, s2)
        qsem = (q0, q1, q2)

        pltpu.sync_copy(z_hbm, acc.at[pl.ds(sid * rpt, rpt)])
        pltpu.sync_copy(zr_hbm, rs.at[pl.ds(sid * rpt, rpt)])
        pltpu.sync_copy(zi_hbm, ziv)
        for b in range(3):
            pltpu.sync_copy(z_hbm.at[pl.ds(0, C)], rows[b])
            pltpu.sync_copy(zr_hbm.at[pl.ds(0, C)], ev[b])
        plsc.subcore_barrier()
        for b in range(3):
            pltpu.async_copy(rows[b], acc.at[ziv], ssem[b], add=True)
            pltpu.async_copy(ev[b], rs.at[ziv], qsem[b], add=True)

        def drain_scatter(b):
            pltpu.make_async_copy(rows[b], acc.at[ziv], ssem[b]).wait()
            pltpu.make_async_copy(ev[b], rs.at[ziv], qsem[b]).wait()

        def issue_gather(m, b):
            pltpu.async_copy(h_hbm.at[dstb.at[m]], rows[b], gsem[b])
            pltpu.async_copy(sl_hbm.at[srcb.at[m]], slv[b], gsem[b])
            pltpu.async_copy(sr_hbm.at[dstb.at[m]], srv[b], gsem[b])

        def process(m, b, prefetch):
            if prefetch:
                p = (b + 2) % 3
                drain_scatter(p)
                issue_gather(m + 2, p)
            pltpu.make_async_copy(h_hbm.at[dstb.at[m]], rows[b], gsem[b]).wait()
            pltpu.make_async_copy(sl_hbm.at[srcb.at[m]], slv[b], gsem[b]).wait()
            pltpu.make_async_copy(sr_hbm.at[dstb.at[m]], srv[b], gsem[b]).wait()
            for g in range(C // 16):
                lg = slv[b][pl.ds(g * 16, 16)] + srv[b][pl.ds(g * 16, 16)]
                lr = jnp.where(lg > 0, lg, lg * ALPHA)
                ev[b][pl.ds(g * 16, 16)] = jnp.exp(-lr)
            buf = rows[b]
            evb = ev[b]

            def scale(k, carry2):
                es = plsc.load_gather(evb, [jnp.full((16,), k, jnp.int32)])
                for j in range(F // 16):
                    buf[k, pl.ds(j * 16, 16)] = buf[k, pl.ds(j * 16, 16)] * es
                return carry2

            lax.fori_loop(0, C, scale, 0, unroll=8)
            pltpu.async_copy(rows[b], acc.at[srcb.at[m]], ssem[b], add=True)
            pltpu.async_copy(ev[b], rs.at[srcb.at[m]], qsem[b], add=True)

        def ib_body(ib, carry):
            rbase = wid * cpw + ib * IB
            pltpu.sync_copy(src_hbm.at[pl.ds(rbase, IB)], srcb)
            pltpu.sync_copy(dst_hbm.at[pl.ds(rbase, IB)], dstb)
            drain_scatter(0)
            issue_gather(0, 0)
            drain_scatter(1)
            issue_gather(1, 1)

            def trip(j, carry2):
                m0 = 3 * j
                process(m0, 0, True)
                process(m0 + 1, 1, True)
                process(m0 + 2, 2, True)
                return carry2

            lax.fori_loop(0, ntrip, trip, 0)
            process(IB - 4, 0, True)
            process(IB - 3, 1, True)
            process(IB - 2, 2, False)
            process(IB - 1, 0, False)
            return carry

        lax.fori_loop(0, nib, ib_body, 0)
        for b in range(3):
            drain_scatter(b)
        plsc.subcore_barrier()
        for k in range(rpt // C):
            pltpu.sync_copy(acc.at[pl.ds(sid * rpt + k * C, C)],
                            out_hbm.at[cid, pl.ds(sid * rpt + k * C, C)])
        pltpu.sync_copy(rs.at[pl.ds(sid * rpt, rpt)],
                        outrs_hbm.at[cid, pl.ds(sid * rpt, rpt)])

    run = pl.kernel(
        body,
        out_type=(jax.ShapeDtypeStruct((2, N_PAD, F), jnp.float32),
                  jax.ShapeDtypeStruct((2, N_PAD), jnp.float32)),
        mesh=_MESH,
        scratch_types=[
            pltpu.VMEM((IB, C), jnp.int32),
            pltpu.VMEM((IB, C), jnp.int32),
            pltpu.VMEM((C, F), jnp.float32),
            pltpu.VMEM((C, F), jnp.float32),
            pltpu.VMEM((C, F), jnp.float32),
            pltpu.VMEM((C,), jnp.float32),
            pltpu.VMEM((C,), jnp.float32),
            pltpu.VMEM((C,), jnp.float32),
            pltpu.VMEM((C,), jnp.float32),
            pltpu.VMEM((C,), jnp.float32),
            pltpu.VMEM((C,), jnp.float32),
            pltpu.VMEM((C,), jnp.float32),
            pltpu.VMEM((C,), jnp.float32),
            pltpu.VMEM((C,), jnp.float32),
            pltpu.VMEM((C,), jnp.int32),
            pltpu.VMEM_SHARED((N_PAD, F), jnp.float32),
            pltpu.VMEM_SHARED((N_PAD,), jnp.float32),
        ] + [pltpu.SemaphoreType.DMA] * 9,
        compiler_params=pltpu.CompilerParams(
            needs_layout_passes=False, use_tc_tiling_on_sc=False),
    )
    return run(h_mat, sl, sr, src2, dst2, zrow, zrs, zidx)


def kernel(x, edges, W_lin, b_lin, W_heads, a_heads, W_end, a_end):
    n = x.shape[0]
    src2 = edges[0].reshape(-1, C)
    dst2 = edges[1].reshape(-1, C)
    zrow = jnp.zeros((N_PAD // 16, F), jnp.float32)
    zrs = jnp.zeros((N_PAD // 16,), jnp.float32)
    zidx = jnp.zeros((C,), jnp.int32)

    h, h0, h1, h2_, h3 = _proj1(x, W_lin, b_lin, W_heads)
    slsr = _avec1(h, W_heads, a_heads)
    parts = []
    rsums = []
    for i, hx in enumerate((h0, h1, h2_, h3)):
        p, r = _sc_pass(hx, slsr[i], slsr[N_HEADS + i], src2, dst2,
                        zrow, zrs, zidx)
        parts.append(p)
        rsums.append(r.reshape(2, N_PAD, 1))
    h2 = _mid(parts, rsums, W_end, n)
    slsr2 = _avec2(h2, a_end)
    p2, r2 = _sc_pass(h2, slsr2[0], slsr2[1], src2, dst2, zrow, zrs, zidx)
    return _final(p2, r2.reshape(2, N_PAD, 1), n)


# speedup vs baseline: 25.4645x; 1.0214x over previous
import jax
import jax.numpy as jnp
from jax import lax
from jax.experimental import pallas as pl
from jax.experimental.pallas import tpu as pltpu
from jax.experimental.pallas import tpu_sc as plsc

ALPHA = 0.2
N_HEADS = 4
F = 128
N_PAD = 10240
C = 80
NW = 32
IB = 25


_BN = 2048


def _proj1_body(x_ref, wl_ref, bl_ref, wh_ref, ah_ref,
                hx0, hx1, hx2, hx3, slsr_ref):
    h = x_ref[...] @ wl_ref[...] + bl_ref[...][None, :]
    outs = (hx0, hx1, hx2, hx3)
    for i in range(N_HEADS):
        hi = h @ wh_ref[i]
        outs[i][...] = hi
        a = ah_ref[i, 0, :]
        slsr_ref[i, :] = jnp.sum(hi * a[:F][None, :], axis=1)
        slsr_ref[N_HEADS + i, :] = jnp.sum(hi * a[F:][None, :], axis=1)


def _proj1(x_pad, w_lin, b_lin, w_heads, a_heads):
    grid = (N_PAD // _BN,)
    return pl.pallas_call(
        _proj1_body,
        grid=grid,
        in_specs=[
            pl.BlockSpec((_BN, F), lambda i: (i, 0)),
            pl.BlockSpec((F, F), lambda i: (0, 0)),
            pl.BlockSpec((F,), lambda i: (0,)),
            pl.BlockSpec((N_HEADS, F, F), lambda i: (0, 0, 0)),
            pl.BlockSpec((N_HEADS, 1, 2 * F), lambda i: (0, 0, 0)),
        ],
        out_specs=[pl.BlockSpec((_BN, F), lambda i: (i, 0))
                   for _ in range(N_HEADS)]
        + [pl.BlockSpec((2 * N_HEADS, _BN), lambda i: (0, i))],
        out_shape=[jax.ShapeDtypeStruct((N_PAD, F), jnp.float32)
                   for _ in range(N_HEADS)]
        + [jax.ShapeDtypeStruct((2 * N_HEADS, N_PAD), jnp.float32)],
    )(x_pad, w_lin, b_lin, w_heads, a_heads)


def _mid_body(p0, p1, p2, p3, r0, r1, r2, r3, we_ref, ae_ref,
              h2_ref, slsr_ref):
    cols = []
    for p, r in zip((p0, p1, p2, p3), (r0, r1, r2, r3)):
        num = p[0] + p[1]
        den = r[0, :, 0] + r[1, :, 0]
        hp = num / (den[:, None] + 1e-16)
        cols.append(jnp.where(hp > 0, hp, jnp.exp(jnp.minimum(hp, 0.0)) - 1.0))
    hcat = jnp.concatenate(cols, axis=1)
    h2 = hcat @ we_ref[...]
    h2_ref[...] = h2
    a = ae_ref[0]
    slsr_ref[0, :] = jnp.sum(h2 * a[:F][None, :], axis=1)
    slsr_ref[1, :] = jnp.sum(h2 * a[F:][None, :], axis=1)


def _mid(parts, rsums, w_end, a_end):
    grid = (N_PAD // _BN,)
    pspec = pl.BlockSpec((2, _BN, F), lambda i: (0, i, 0))
    rspec = pl.BlockSpec((2, _BN, 1), lambda i: (0, i, 0))
    return pl.pallas_call(
        _mid_body,
        grid=grid,
        in_specs=[pspec] * 4 + [rspec] * 4
        + [pl.BlockSpec((4 * F, F), lambda i: (0, 0)),
           pl.BlockSpec((1, 2 * F), lambda i: (0, 0))],
        out_specs=[pl.BlockSpec((_BN, F), lambda i: (i, 0)),
                   pl.BlockSpec((2, _BN), lambda i: (0, i))],
        out_shape=[jax.ShapeDtypeStruct((N_PAD, F), jnp.float32),
                   jax.ShapeDtypeStruct((2, N_PAD), jnp.float32)],
    )(*parts, *rsums, w_end, a_end)


def _final_body(p_ref, r_ref, out_ref):
    num = p_ref[0] + p_ref[1]
    den = r_ref[0, :, 0] + r_ref[1, :, 0]
    hp = num / (den[:, None] + 1e-16)
    m = jnp.max(hp, axis=1, keepdims=True)
    ex = jnp.exp(hp - m)
    out_ref[...] = ex / jnp.sum(ex, axis=1, keepdims=True)


def _final(part, rsum):
    grid = (N_PAD // _BN,)
    return pl.pallas_call(
        _final_body,
        grid=grid,
        in_specs=[pl.BlockSpec((2, _BN, F), lambda i: (0, i, 0)),
                  pl.BlockSpec((2, _BN, 1), lambda i: (0, i, 0))],
        out_specs=pl.BlockSpec((_BN, F), lambda i: (i, 0)),
        out_shape=jax.ShapeDtypeStruct((N_PAD, F), jnp.float32),
    )(part, rsum)


_MESH = plsc.VectorSubcoreMesh(
    core_axis_name="c", subcore_axis_name="s", num_cores=2, num_subcores=16)


def _sc_pass(h_mat, sl, sr, src2, dst2, zrow, zrs, zidx):
    n = h_mat.shape[0]
    e = src2.shape[0] * src2.shape[1]
    cpw = e // (NW * C)
    nib = cpw // IB
    rpt = N_PAD // 16
    ntrip = (IB - 4) // 3

    def body(h_hbm, sl_hbm, sr_hbm, src_hbm, dst_hbm, z_hbm, zr_hbm, zi_hbm,
             out_hbm, outrs_hbm,
             srcb, dstb, r0, r1, r2, sl0, sl1, sl2, sr0, sr1, sr2,
             e0, e1, e2, ziv, acc, rs,
             g0, g1, g2, s0, s1, s2, q0, q1, q2):
        cid = lax.axis_index("c")
        sid = lax.axis_index("s")
        wid = cid * 16 + sid
        rows = (r0, r1, r2)
        slv = (sl0, sl1, sl2)
        srv = (sr0, sr1, sr2)
        ev = (e0, e1, e2)
        gsem = (g0, g1, g2)
        ssem = (s0, s1, s2)
        qsem = (q0, q1, q2)

        pltpu.sync_copy(z_hbm, acc.at[pl.ds(sid * rpt, rpt)])
        pltpu.sync_copy(zr_hbm, rs.at[pl.ds(sid * rpt, rpt)])
        pltpu.sync_copy(zi_hbm, ziv)
        for b in range(3):
            pltpu.sync_copy(z_hbm.at[pl.ds(0, C)], rows[b])
            pltpu.sync_copy(zr_hbm.at[pl.ds(0, C)], ev[b])
        plsc.subcore_barrier()
        for b in range(3):
            pltpu.async_copy(rows[b], acc.at[ziv], ssem[b], add=True)
            pltpu.async_copy(ev[b], rs.at[ziv], qsem[b], add=True)

        def drain_scatter(b):
            pltpu.make_async_copy(rows[b], acc.at[ziv], ssem[b]).wait()
            pltpu.make_async_copy(ev[b], rs.at[ziv], qsem[b]).wait()

        def issue_gather(m, b):
            pltpu.async_copy(h_hbm.at[dstb.at[m]], rows[b], gsem[b])
            pltpu.async_copy(sl_hbm.at[srcb.at[m]], slv[b], gsem[b])
            pltpu.async_copy(sr_hbm.at[dstb.at[m]], srv[b], gsem[b])

        def process(m, b, prefetch):
            if prefetch:
                p = (b + 2) % 3
                drain_scatter(p)
                issue_gather(m + 2, p)
            pltpu.make_async_copy(h_hbm.at[dstb.at[m]], rows[b], gsem[b]).wait()
            pltpu.make_async_copy(sl_hbm.at[srcb.at[m]], slv[b], gsem[b]).wait()
            pltpu.make_async_copy(sr_hbm.at[dstb.at[m]], srv[b], gsem[b]).wait()
            for g in range(C // 16):
                lg = slv[b][pl.ds(g * 16, 16)] + srv[b][pl.ds(g * 16, 16)]
                lr = jnp.where(lg > 0, lg, lg * ALPHA)
                ev[b][pl.ds(g * 16, 16)] = jnp.exp(-lr)
            buf = rows[b]
            evb = ev[b]

            def scale(k, carry2):
                es = plsc.load_gather(evb, [jnp.full((16,), k, jnp.int32)])
                for j in range(F // 16):
                    buf[k, pl.ds(j * 16, 16)] = buf[k, pl.ds(j * 16, 16)] * es
                return carry2

            lax.fori_loop(0, C, scale, 0, unroll=8)
            pltpu.async_copy(rows[b], acc.at[srcb.at[m]], ssem[b], add=True)
            pltpu.async_copy(ev[b], rs.at[srcb.at[m]], qsem[b], add=True)

        def ib_body(ib, carry):
            rbase = wid * cpw + ib * IB
            pltpu.sync_copy(src_hbm.at[pl.ds(rbase, IB)], srcb)
            pltpu.sync_copy(dst_hbm.at[pl.ds(rbase, IB)], dstb)
            drain_scatter(0)
            issue_gather(0, 0)
            drain_scatter(1)
            issue_gather(1, 1)

            def trip(j, carry2):
                m0 = 3 * j
                process(m0, 0, True)
                process(m0 + 1, 1, True)
                process(m0 + 2, 2, True)
                return carry2

            lax.fori_loop(0, ntrip, trip, 0)
            process(IB - 4, 0, True)
            process(IB - 3, 1, True)
            process(IB - 2, 2, False)
            process(IB - 1, 0, False)
            return carry

        lax.fori_loop(0, nib, ib_body, 0)
        for b in range(3):
            drain_scatter(b)
        plsc.subcore_barrier()
        for k in range(rpt // C):
            pltpu.sync_copy(acc.at[pl.ds(sid * rpt + k * C, C)],
                            out_hbm.at[cid, pl.ds(sid * rpt + k * C, C)])
        pltpu.sync_copy(rs.at[pl.ds(sid * rpt, rpt)],
                        outrs_hbm.at[cid, pl.ds(sid * rpt, rpt)])

    run = pl.kernel(
        body,
        out_type=(jax.ShapeDtypeStruct((2, N_PAD, F), jnp.float32),
                  jax.ShapeDtypeStruct((2, N_PAD), jnp.float32)),
        mesh=_MESH,
        scratch_types=[
            pltpu.VMEM((IB, C), jnp.int32),
            pltpu.VMEM((IB, C), jnp.int32),
            pltpu.VMEM((C, F), jnp.float32),
            pltpu.VMEM((C, F), jnp.float32),
            pltpu.VMEM((C, F), jnp.float32),
            pltpu.VMEM((C,), jnp.float32),
            pltpu.VMEM((C,), jnp.float32),
            pltpu.VMEM((C,), jnp.float32),
            pltpu.VMEM((C,), jnp.float32),
            pltpu.VMEM((C,), jnp.float32),
            pltpu.VMEM((C,), jnp.float32),
            pltpu.VMEM((C,), jnp.float32),
            pltpu.VMEM((C,), jnp.float32),
            pltpu.VMEM((C,), jnp.float32),
            pltpu.VMEM((C,), jnp.int32),
            pltpu.VMEM_SHARED((N_PAD, F), jnp.float32),
            pltpu.VMEM_SHARED((N_PAD,), jnp.float32),
        ] + [pltpu.SemaphoreType.DMA] * 9,
        compiler_params=pltpu.CompilerParams(
            needs_layout_passes=False, use_tc_tiling_on_sc=False),
    )
    return run(h_mat, sl, sr, src2, dst2, zrow, zrs, zidx)


def kernel(x, edges, W_lin, b_lin, W_heads, a_heads, W_end, a_end):
    n = x.shape[0]
    src2 = edges[0].reshape(-1, C)
    dst2 = edges[1].reshape(-1, C)
    zrow = jnp.zeros((N_PAD // 16, F), jnp.float32)
    zrs = jnp.zeros((N_PAD // 16,), jnp.float32)
    zidx = jnp.zeros((C,), jnp.int32)

    x_pad = jnp.concatenate(
        [x, jnp.zeros((N_PAD - n, F), jnp.float32)], axis=0)
    h0, h1, h2_, h3, slsr = _proj1(x_pad, W_lin, b_lin, W_heads, a_heads)
    parts = []
    rsums = []
    for i, hx in enumerate((h0, h1, h2_, h3)):
        p, r = _sc_pass(hx, slsr[i], slsr[N_HEADS + i], src2, dst2,
                        zrow, zrs, zidx)
        parts.append(p)
        rsums.append(r.reshape(2, N_PAD, 1))
    h2, slsr2 = _mid(parts, rsums, W_end, a_end)
    p2, r2 = _sc_pass(h2, slsr2[0], slsr2[1], src2, dst2, zrow, zrs, zidx)
    return _final(p2, r2.reshape(2, N_PAD, 1))[:n]
